# Initial kernel scaffold; baseline (speedup 1.0000x reference)
import jax
import jax.numpy as jnp
from jax.experimental import pallas as pl


def kernel(in_degree, out_degree, table1, table2):
    raise NotImplementedError("write your pallas kernel here")


# speedup vs baseline: 1.1848x; 1.1848x over previous
import functools

import jax
import jax.numpy as jnp
from jax import lax
from jax.experimental import pallas as pl
from jax.experimental.pallas import tpu as pltpu
from jax.experimental.pallas import tpu_sc as plsc

MAXD = 64
ROWS = MAXD + 1
D = 512
N = 100000
NC, NS, LANES = 2, 16, 16
NW = NC * NS
CHUNK = 125
CHUNK_PAD = 128
NCHUNK = N // CHUNK
PER_W = NCHUNK // NW


def _build_combo(table1, table2):
    def body(t1_ref, t2_ref, out_ref):
        out_ref[...] = t1_ref[...][:, None, :] + t2_ref[...][None, :, :]

    out = pl.pallas_call(
        body,
        out_shape=jax.ShapeDtypeStruct((ROWS, ROWS, D), jnp.float32),
    )(table1, table2)
    return out.reshape(ROWS * ROWS, D)


def _sc_gather(combo, ind, outd):
    mesh = plsc.VectorSubcoreMesh(core_axis_name="c", subcore_axis_name="s")

    @functools.partial(
        pl.kernel,
        out_type=jax.ShapeDtypeStruct((N, D), jnp.float32),
        mesh=mesh,
        compiler_params=pltpu.CompilerParams(use_tc_tiling_on_sc=False),
        scratch_types=[
            pltpu.VMEM((CHUNK_PAD,), jnp.int32),
            pltpu.VMEM((CHUNK_PAD,), jnp.int32),
            pltpu.VMEM((CHUNK_PAD,), jnp.int32),
            pltpu.VMEM((CHUNK_PAD, D), jnp.float32),
            pltpu.SemaphoreType.DMA,
        ],
    )
    def k(combo_hbm, ind_hbm, outd_hbm, out_hbm, i1_v, i2_v, idx_v, rows_v, sem):
        wid = lax.axis_index("s") * NC + lax.axis_index("c")

        @pl.loop(0, PER_W)
        def _(j):
            c = wid * PER_W + j
            pltpu.sync_copy(ind_hbm.at[c], i1_v)
            pltpu.sync_copy(outd_hbm.at[c], i2_v)

            @pl.loop(0, CHUNK_PAD, step=LANES)
            def _(k0):
                a = i1_v[pl.ds(k0, LANES)]
                b = i2_v[pl.ds(k0, LANES)]
                a = jnp.minimum(jnp.maximum(a, 0), MAXD)
                b = jnp.minimum(jnp.maximum(b, 0), MAXD)
                idx_v[pl.ds(k0, LANES)] = a * ROWS + b

            pltpu.async_copy(combo_hbm.at[idx_v], rows_v, sem).wait()
            pltpu.sync_copy(rows_v.at[pl.ds(0, CHUNK)],
                            out_hbm.at[pl.ds(c * CHUNK, CHUNK)])

    return k(combo, ind, outd)


def kernel(in_degree, out_degree, table1, table2):
    combo = _build_combo(table1, table2)
    pad = ((0, 0), (0, CHUNK_PAD - CHUNK))
    ind = jnp.pad(in_degree.reshape(NCHUNK, CHUNK), pad)
    outd = jnp.pad(out_degree.reshape(NCHUNK, CHUNK), pad)
    return _sc_gather(combo, ind, outd)


# speedup vs baseline: 1.7576x; 1.4835x over previous
import functools

import jax
import jax.numpy as jnp
from jax import lax
from jax.experimental import pallas as pl
from jax.experimental.pallas import tpu as pltpu
from jax.experimental.pallas import tpu_sc as plsc

MAXD = 64
ROWS = MAXD + 1
D = 512
N = 100000
NC, NS, LANES = 2, 16, 16
NW = NC * NS
CHUNK = 120
CHUNK_PAD = 128
PER_W = 26
N_MAIN = NW * PER_W * CHUNK
TAIL_LEN = 80


def _build_combo(table1, table2):
    def body(t1_ref, t2_ref, out_ref):
        out_ref[...] = t1_ref[...][:, None, :] + t2_ref[...][None, :, :]

    out = pl.pallas_call(
        body,
        out_shape=jax.ShapeDtypeStruct((ROWS, ROWS, D), jnp.float32),
    )(table1, table2)
    return out.reshape(ROWS * ROWS, D)


def _prep_idx(x):
    main = jnp.pad(x[:N_MAIN].reshape(N_MAIN // CHUNK, CHUNK),
                   ((0, 0), (0, CHUNK_PAD - CHUNK)))
    tail = jnp.pad(x[N_MAIN:].reshape(2, TAIL_LEN),
                   ((0, 0), (0, CHUNK_PAD - TAIL_LEN)))
    return jnp.concatenate([main, tail], axis=0)


def _sc_gather(combo, ind, outd):
    mesh = plsc.VectorSubcoreMesh(core_axis_name="c", subcore_axis_name="s")

    @functools.partial(
        pl.kernel,
        out_type=jax.ShapeDtypeStruct((N, D), jnp.float32),
        mesh=mesh,
        compiler_params=pltpu.CompilerParams(use_tc_tiling_on_sc=False),
        scratch_types=[
            pltpu.VMEM((2, CHUNK_PAD), jnp.int32),
            pltpu.VMEM((2, CHUNK_PAD), jnp.int32),
            pltpu.VMEM((2, CHUNK_PAD), jnp.int32),
            pltpu.VMEM((2, CHUNK, D), jnp.float32),
            pltpu.SemaphoreType.DMA,
            pltpu.SemaphoreType.DMA,
            pltpu.SemaphoreType.DMA,
            pltpu.SemaphoreType.DMA,
        ],
    )
    def k(combo_hbm, ind_hbm, outd_hbm, out_hbm,
          i1_v, i2_v, idxf_v, rows_v, sg0, sg1, si1, si2):
        wid = lax.axis_index("s") * NC + lax.axis_index("c")
        c0 = wid * PER_W

        def load_idx_async(row, b):
            h1 = pltpu.async_copy(ind_hbm.at[row], i1_v.at[b], si1)
            h2 = pltpu.async_copy(outd_hbm.at[row], i2_v.at[b], si2)
            return (h1, h2)

        def compute_idx(b):
            for k0 in range(0, CHUNK_PAD, LANES):
                a = i1_v[b, pl.ds(k0, LANES)]
                bb = i2_v[b, pl.ds(k0, LANES)]
                a = jnp.minimum(jnp.maximum(a, 0), MAXD)
                bb = jnp.minimum(jnp.maximum(bb, 0), MAXD)
                idxf_v[b, pl.ds(k0, LANES)] = a * ROWS + bb

        def start_gather(b, count):
            sem = sg0 if b == 0 else sg1
            return pltpu.async_copy(
                combo_hbm.at[idxf_v.at[b, pl.ds(0, count)]],
                rows_v.at[b].at[pl.ds(0, count)], sem)

        pltpu.sync_copy(ind_hbm.at[c0], i1_v.at[0])
        pltpu.sync_copy(outd_hbm.at[c0], i2_v.at[0])
        compute_idx(0)
        gathers = {0: start_gather(0, CHUNK)}
        pending_idx = load_idx_async(c0 + 1, 1)

        for j in range(PER_W):
            b = j % 2
            nb = 1 - b
            if j + 1 < PER_W:
                for h in pending_idx:
                    h.wait()
                compute_idx(nb)
            gathers[b].wait()
            if j + 1 < PER_W:
                gathers[nb] = start_gather(nb, CHUNK)
                if j + 2 < PER_W:
                    pending_idx = load_idx_async(c0 + j + 2, b)
            pltpu.sync_copy(rows_v.at[b],
                            out_hbm.at[pl.ds((c0 + j) * CHUNK, CHUNK)])

        num_main_chunks = NW * PER_W
        for t in range(2):
            @pl.when(wid == t)
            def _():
                pltpu.sync_copy(ind_hbm.at[num_main_chunks + t], i1_v.at[0])
                pltpu.sync_copy(outd_hbm.at[num_main_chunks + t], i2_v.at[0])
                compute_idx(0)
                start_gather(0, TAIL_LEN).wait()
                pltpu.sync_copy(
                    rows_v.at[0].at[pl.ds(0, TAIL_LEN)],
                    out_hbm.at[pl.ds(N_MAIN + t * TAIL_LEN, TAIL_LEN)])

    return k(combo, ind, outd)


def kernel(in_degree, out_degree, table1, table2):
    combo = _build_combo(table1, table2)
    return _sc_gather(combo, _prep_idx(in_degree), _prep_idx(out_degree))


# speedup vs baseline: 3.6892x; 2.0989x over previous
import functools

import jax
import jax.numpy as jnp
from jax import lax
from jax.experimental import pallas as pl
from jax.experimental.pallas import tpu as pltpu
from jax.experimental.pallas import tpu_sc as plsc

MAXD = 64
ROWS = MAXD + 1
D = 512
N = 100000
NC, NS, LANES = 2, 16, 16
NW = NC * NS
CHUNK = 120
CHUNK_PAD = 128
PER_W = 26
N_MAIN = NW * PER_W * CHUNK
TAIL_LEN = 80


def _build_combo(table1, table2):
    def body(t1_ref, t2_ref, out_ref):
        out_ref[...] = t1_ref[...][:, None, :] + t2_ref[...][None, :, :]

    out = pl.pallas_call(
        body,
        out_shape=jax.ShapeDtypeStruct((ROWS, ROWS, D), jnp.float32),
    )(table1, table2)
    return out.reshape(ROWS * ROWS, D)


def _prep_idx(x):
    main = jnp.pad(x[:N_MAIN].reshape(N_MAIN // CHUNK, CHUNK),
                   ((0, 0), (0, CHUNK_PAD - CHUNK)))
    tail = jnp.pad(x[N_MAIN:].reshape(2, TAIL_LEN),
                   ((0, 0), (0, CHUNK_PAD - TAIL_LEN)))
    return jnp.concatenate([main, tail], axis=0)


def _sc_gather(combo, ind, outd):
    mesh = plsc.VectorSubcoreMesh(core_axis_name="c", subcore_axis_name="s")

    @functools.partial(
        pl.kernel,
        out_type=jax.ShapeDtypeStruct((N, D), jnp.float32),
        mesh=mesh,
        compiler_params=pltpu.CompilerParams(use_tc_tiling_on_sc=True),
        scratch_types=[
            pltpu.VMEM((2, CHUNK_PAD), jnp.int32),
            pltpu.VMEM((2, CHUNK_PAD), jnp.int32),
            pltpu.VMEM((2, CHUNK_PAD), jnp.int32),
            pltpu.VMEM((2, CHUNK, D), jnp.float32),
            pltpu.SemaphoreType.DMA,
            pltpu.SemaphoreType.DMA,
            pltpu.SemaphoreType.DMA,
            pltpu.SemaphoreType.DMA,
        ],
    )
    def k(combo_hbm, ind_hbm, outd_hbm, out_hbm,
          i1_v, i2_v, idxf_v, rows_v, sg0, sg1, si1, si2):
        wid = lax.axis_index("s") * NC + lax.axis_index("c")
        c0 = wid * PER_W

        def load_idx_async(row, b):
            h1 = pltpu.async_copy(ind_hbm.at[row], i1_v.at[b], si1)
            h2 = pltpu.async_copy(outd_hbm.at[row], i2_v.at[b], si2)
            return (h1, h2)

        def compute_idx(b):
            for k0 in range(0, CHUNK_PAD, LANES):
                a = i1_v[b, pl.ds(k0, LANES)]
                bb = i2_v[b, pl.ds(k0, LANES)]
                a = jnp.minimum(jnp.maximum(a, 0), MAXD)
                bb = jnp.minimum(jnp.maximum(bb, 0), MAXD)
                idxf_v[b, pl.ds(k0, LANES)] = a * ROWS + bb

        def start_gather(b, count):
            sem = sg0 if b == 0 else sg1
            return pltpu.async_copy(
                combo_hbm.at[idxf_v.at[b, pl.ds(0, count)]],
                rows_v.at[b].at[pl.ds(0, count)], sem)

        pltpu.sync_copy(ind_hbm.at[c0], i1_v.at[0])
        pltpu.sync_copy(outd_hbm.at[c0], i2_v.at[0])
        compute_idx(0)
        gathers = {0: start_gather(0, CHUNK)}
        pending_idx = load_idx_async(c0 + 1, 1)

        for j in range(PER_W):
            b = j % 2
            nb = 1 - b
            if j + 1 < PER_W:
                for h in pending_idx:
                    h.wait()
                compute_idx(nb)
            gathers[b].wait()
            if j + 1 < PER_W:
                gathers[nb] = start_gather(nb, CHUNK)
                if j + 2 < PER_W:
                    pending_idx = load_idx_async(c0 + j + 2, b)
            pltpu.sync_copy(rows_v.at[b],
                            out_hbm.at[pl.ds((c0 + j) * CHUNK, CHUNK)])

        num_main_chunks = NW * PER_W
        for t in range(2):
            @pl.when(wid == t)
            def _():
                pltpu.sync_copy(ind_hbm.at[num_main_chunks + t], i1_v.at[0])
                pltpu.sync_copy(outd_hbm.at[num_main_chunks + t], i2_v.at[0])
                compute_idx(0)
                start_gather(0, TAIL_LEN).wait()
                pltpu.sync_copy(
                    rows_v.at[0].at[pl.ds(0, TAIL_LEN)],
                    out_hbm.at[pl.ds(N_MAIN + t * TAIL_LEN, TAIL_LEN)])

    return k(combo, ind, outd)


def kernel(in_degree, out_degree, table1, table2):
    combo = _build_combo(table1, table2)
    return _sc_gather(combo, _prep_idx(in_degree), _prep_idx(out_degree))


# speedup vs baseline: 3.8852x; 1.0531x over previous
import functools

import jax
import jax.numpy as jnp
from jax import lax
from jax.experimental import pallas as pl
from jax.experimental.pallas import tpu as pltpu
from jax.experimental.pallas import tpu_sc as plsc

MAXD = 64
ROWS = MAXD + 1
ROWS_PAD = 72
D = 512
N = 100000
NC, NS, LANES = 2, 16, 16
NW = NC * NS
CHUNK = 120
CHUNK_PAD = 128
PER_W = 26
N_MAIN = NW * PER_W * CHUNK
TAIL_LEN = 80


def _build_combo(table1, table2):
    def body(t1_ref, t2_ref, out_ref):
        t2x = jnp.concatenate(
            [t2_ref[...], jnp.zeros((ROWS_PAD - ROWS, D), jnp.float32)], axis=0)
        out_ref[...] = t1_ref[...][:, None, :] + t2x[None, :, :]

    out = pl.pallas_call(
        body,
        out_shape=jax.ShapeDtypeStruct((ROWS, ROWS_PAD, D), jnp.float32),
    )(table1, table2)
    return out.reshape(ROWS * ROWS_PAD, D)


def _sc_gather(combo, ind, outd):
    mesh = plsc.VectorSubcoreMesh(core_axis_name="c", subcore_axis_name="s")

    @functools.partial(
        pl.kernel,
        out_type=jax.ShapeDtypeStruct((N, D), jnp.float32),
        mesh=mesh,
        compiler_params=pltpu.CompilerParams(use_tc_tiling_on_sc=True),
        scratch_types=[
            pltpu.VMEM((2, CHUNK_PAD), jnp.int32),
            pltpu.VMEM((2, CHUNK_PAD), jnp.int32),
            pltpu.VMEM((2, CHUNK_PAD), jnp.int32),
            pltpu.VMEM((2, CHUNK, D), jnp.float32),
            pltpu.SemaphoreType.DMA,
            pltpu.SemaphoreType.DMA,
            pltpu.SemaphoreType.DMA,
            pltpu.SemaphoreType.DMA,
            pltpu.SemaphoreType.DMA,
            pltpu.SemaphoreType.DMA,
        ],
    )
    def k(combo_hbm, ind_hbm, outd_hbm, out_hbm,
          i1_v, i2_v, idxf_v, rows_v, sg0, sg1, si1, si2, sw0, sw1):
        wid = lax.axis_index("s") * NC + lax.axis_index("c")
        c0 = wid * PER_W

        def load_idx_async(off, b, count):
            h1 = pltpu.async_copy(ind_hbm.at[pl.ds(off, count)],
                                  i1_v.at[b, pl.ds(0, count)], si1)
            h2 = pltpu.async_copy(outd_hbm.at[pl.ds(off, count)],
                                  i2_v.at[b, pl.ds(0, count)], si2)
            return (h1, h2)

        def compute_idx(b):
            for k0 in range(0, CHUNK_PAD, LANES):
                a = i1_v[b, pl.ds(k0, LANES)]
                bb = i2_v[b, pl.ds(k0, LANES)]
                a = jnp.minimum(jnp.maximum(a, 0), MAXD)
                bb = jnp.minimum(jnp.maximum(bb, 0), MAXD)
                idxf_v[b, pl.ds(k0, LANES)] = a * ROWS_PAD + bb

        def start_gather(b, count):
            sem = sg0 if b == 0 else sg1
            return pltpu.async_copy(
                combo_hbm.at[idxf_v.at[b, pl.ds(0, count)]],
                rows_v.at[b].at[pl.ds(0, count)], sem)

        def start_write(j, b):
            sem = sw0 if b == 0 else sw1
            return pltpu.async_copy(
                rows_v.at[b], out_hbm.at[pl.ds((c0 + j) * CHUNK, CHUNK)], sem)

        for h in load_idx_async(c0 * CHUNK, 0, CHUNK):
            h.wait()
        compute_idx(0)
        gathers = {0: start_gather(0, CHUNK)}
        writes = {}
        pending_idx = load_idx_async((c0 + 1) * CHUNK, 1, CHUNK)

        for j in range(PER_W):
            b = j % 2
            nb = 1 - b
            if j + 1 < PER_W:
                for h in pending_idx:
                    h.wait()
                compute_idx(nb)
                if nb in writes:
                    writes.pop(nb).wait()
            gathers.pop(b).wait()
            if j + 1 < PER_W:
                gathers[nb] = start_gather(nb, CHUNK)
                if j + 2 < PER_W:
                    pending_idx = load_idx_async((c0 + j + 2) * CHUNK, b, CHUNK)
            writes[b] = start_write(j, b)
        for b in list(writes):
            writes.pop(b).wait()

        for t in range(2):
            @pl.when(wid == t)
            def _():
                for h in load_idx_async(N_MAIN + t * TAIL_LEN, 0, TAIL_LEN):
                    h.wait()
                compute_idx(0)
                start_gather(0, TAIL_LEN).wait()
                pltpu.sync_copy(
                    rows_v.at[0].at[pl.ds(0, TAIL_LEN)],
                    out_hbm.at[pl.ds(N_MAIN + t * TAIL_LEN, TAIL_LEN)])

    return k(combo, ind, outd)


def kernel(in_degree, out_degree, table1, table2):
    combo = _build_combo(table1, table2)
    return _sc_gather(combo, in_degree, out_degree)


# speedup vs baseline: 3.9006x; 1.0040x over previous
import functools

import jax
import jax.numpy as jnp
from jax import lax
from jax.experimental import pallas as pl
from jax.experimental.pallas import tpu as pltpu
from jax.experimental.pallas import tpu_sc as plsc

MAXD = 64
ROWS = MAXD + 1
ROWS_PAD = 72
D = 512
N = 100000
NC, NS, LANES = 2, 16, 16
NW = NC * NS
CHUNK = 80
CHUNK_PAD = 128
RING = 3
PER_W = 39
N_MAIN = NW * PER_W * CHUNK
TAIL_LEN = 80


def _build_combo(table1, table2):
    def body(t1_ref, t2_ref, out_ref):
        t2x = jnp.concatenate(
            [t2_ref[...], jnp.zeros((ROWS_PAD - ROWS, D), jnp.float32)], axis=0)
        out_ref[...] = t1_ref[...][:, None, :] + t2x[None, :, :]

    out = pl.pallas_call(
        body,
        out_shape=jax.ShapeDtypeStruct((ROWS, ROWS_PAD, D), jnp.float32),
    )(table1, table2)
    return out.reshape(ROWS * ROWS_PAD, D)


def _sc_gather(combo, ind, outd):
    mesh = plsc.VectorSubcoreMesh(core_axis_name="c", subcore_axis_name="s")

    @functools.partial(
        pl.kernel,
        out_type=jax.ShapeDtypeStruct((N, D), jnp.float32),
        mesh=mesh,
        compiler_params=pltpu.CompilerParams(use_tc_tiling_on_sc=True),
        scratch_types=[
            pltpu.VMEM((RING, CHUNK_PAD), jnp.int32),
            pltpu.VMEM((RING, CHUNK_PAD), jnp.int32),
            pltpu.VMEM((RING, CHUNK_PAD), jnp.int32),
            pltpu.VMEM((RING, CHUNK, D), jnp.float32),
            [pltpu.SemaphoreType.DMA] * RING,
            [pltpu.SemaphoreType.DMA] * RING,
            pltpu.SemaphoreType.DMA,
            pltpu.SemaphoreType.DMA,
        ],
    )
    def k(combo_hbm, ind_hbm, outd_hbm, out_hbm,
          i1_v, i2_v, idxf_v, rows_v, sg, sw, si1, si2):
        wid = lax.axis_index("s") * NC + lax.axis_index("c")
        c0 = wid * PER_W

        def load_idx_async(off, b, count):
            h1 = pltpu.async_copy(ind_hbm.at[pl.ds(off, count)],
                                  i1_v.at[b, pl.ds(0, count)], si1)
            h2 = pltpu.async_copy(outd_hbm.at[pl.ds(off, count)],
                                  i2_v.at[b, pl.ds(0, count)], si2)
            return (h1, h2)

        def compute_idx(b):
            for k0 in range(0, CHUNK_PAD, LANES):
                a = i1_v[b, pl.ds(k0, LANES)]
                bb = i2_v[b, pl.ds(k0, LANES)]
                a = jnp.minimum(jnp.maximum(a, 0), MAXD)
                bb = jnp.minimum(jnp.maximum(bb, 0), MAXD)
                idxf_v[b, pl.ds(k0, LANES)] = a * ROWS_PAD + bb

        def start_gather(b, count):
            return pltpu.async_copy(
                combo_hbm.at[idxf_v.at[b, pl.ds(0, count)]],
                rows_v.at[b].at[pl.ds(0, count)], sg[b])

        def start_write(j, b):
            return pltpu.async_copy(
                rows_v.at[b], out_hbm.at[pl.ds((c0 + j) * CHUNK, CHUNK)], sw[b])

        for h in load_idx_async(c0 * CHUNK, 0, CHUNK):
            h.wait()
        compute_idx(0)
        gathers = {0: start_gather(0, CHUNK)}
        writes = {}
        pending_idx = load_idx_async((c0 + 1) * CHUNK, 1, CHUNK)

        for j in range(PER_W):
            b = j % RING
            nb = (j + 1) % RING
            if j + 1 < PER_W:
                for h in pending_idx:
                    h.wait()
                compute_idx(nb)
                if nb in writes:
                    writes.pop(nb).wait()
            gathers.pop(b).wait()
            if j + 1 < PER_W:
                gathers[nb] = start_gather(nb, CHUNK)
                if j + 2 < PER_W:
                    pending_idx = load_idx_async((c0 + j + 2) * CHUNK,
                                                 (j + 2) % RING, CHUNK)
            writes[b] = start_write(j, b)
        for b in list(writes):
            writes.pop(b).wait()

        for t in range(2):
            @pl.when(wid == t)
            def _():
                for h in load_idx_async(N_MAIN + t * TAIL_LEN, 0, TAIL_LEN):
                    h.wait()
                compute_idx(0)
                start_gather(0, TAIL_LEN).wait()
                pltpu.sync_copy(
                    rows_v.at[0].at[pl.ds(0, TAIL_LEN)],
                    out_hbm.at[pl.ds(N_MAIN + t * TAIL_LEN, TAIL_LEN)])

    return k(combo, ind, outd)


def kernel(in_degree, out_degree, table1, table2):
    combo = _build_combo(table1, table2)
    return _sc_gather(combo, in_degree, out_degree)
